# baseline (speedup 1.0000x reference)
import jax
import jax.numpy as jnp
from jax import lax
from jax.experimental import pallas as pl
from jax.experimental.pallas import tpu as pltpu
from jax.experimental.pallas import tpu_sc as plsc

N_NODES = 10000
D = 128
E = 320000
SG_SIZE = 8
N_SUB = 40000
H = 256
D2 = 2 * D

NC = 2
NS = 16
NW = NC * NS
CH = 80
RPW = E // NW // CH

NB = 50
SEGB = E // NB
SROWS = SEGB + 128
CAPW = 320
TRASH = SEGB


def _make_mesh():
    return plsc.VectorSubcoreMesh(core_axis_name="c", subcore_axis_name="s")


def _lrelu(x):
    return jnp.where(x >= 0, x, x * 0.01)


def _bf(x):
    return x.astype(jnp.bfloat16)


def _dot(a, b):
    return jnp.dot(_bf(a), _bf(b), preferred_element_type=jnp.float32)


def _seg_nodes_body(x_hbm, src_hbm, dst_hbm, zeros_hbm, out_hbm,
                    sidx, didx, rows, acc_sh, sem):
    c = lax.axis_index("c")
    s = lax.axis_index("s")
    w = c * NS + s
    rpt = 640
    pltpu.sync_copy(zeros_hbm, acc_sh.at[pl.ds(s * rpt, rpt)])
    pltpu.sync_copy(src_hbm.at[w], sidx)
    pltpu.sync_copy(dst_hbm.at[w], didx)
    plsc.subcore_barrier()

    def chunk(j, carry):
        pltpu.async_copy(x_hbm.at[sidx.at[j]], rows, sem).wait()
        pltpu.sync_copy(rows, acc_sh.at[didx.at[j]], add=True)
        return carry

    lax.fori_loop(0, RPW, chunk, 0)
    plsc.subcore_barrier()

    @pl.when(s < NS - 1)
    def _():
        pltpu.sync_copy(acc_sh.at[pl.ds(s * rpt, rpt)],
                        out_hbm.at[c, pl.ds(s * rpt, rpt)])

    @pl.when(s == NS - 1)
    def _():
        pltpu.sync_copy(
            acc_sh.at[pl.ds((NS - 1) * rpt, N_NODES - (NS - 1) * rpt)],
            out_hbm.at[c, pl.ds((NS - 1) * rpt, N_NODES - (NS - 1) * rpt)])


def _seg_nodes(x, src3, dst3, zeros):
    f = pl.kernel(
        _seg_nodes_body,
        out_type=jax.ShapeDtypeStruct((NC, N_NODES, D), jnp.float32),
        mesh=_make_mesh(),
        scratch_types=[
            pltpu.VMEM((RPW, CH), jnp.int32),
            pltpu.VMEM((RPW, CH), jnp.int32),
            pltpu.VMEM((CH, D), jnp.float32),
            pltpu.VMEM_SHARED((NS * 640, D), jnp.float32),
            pltpu.SemaphoreType.DMA,
        ],
    )
    return f(x, src3, dst3, zeros)


def _gather_body(tab_hbm, idx_hbm, out_hbm, gidx, rows, sem):
    c = lax.axis_index("c")
    s = lax.axis_index("s")
    w = c * NS + s
    pltpu.sync_copy(idx_hbm.at[w], gidx)

    def chunk(j, carry):
        pltpu.async_copy(tab_hbm.at[gidx.at[j]], rows, sem).wait()
        pltpu.sync_copy(rows, out_hbm.at[pl.ds(w * (RPW * CH) + j * CH, CH)])
        return carry

    lax.fori_loop(0, RPW, chunk, 0)


def _gather_rows(tab, idx3):
    width = tab.shape[-1]
    f = pl.kernel(
        _gather_body,
        out_type=jax.ShapeDtypeStruct((E, width), jnp.float32),
        mesh=_make_mesh(),
        scratch_types=[
            pltpu.VMEM((RPW, CH), jnp.int32),
            pltpu.VMEM((CH, width), jnp.float32),
            pltpu.SemaphoreType.DMA,
        ],
    )
    return f(tab, idx3)


def _seg_big_body(x1_hbm, x2_hbm, bsrc_hbm, bldst_hbm, zeros_hbm,
                  out1_hbm, out2_hbm,
                  sidx, lidx, rows1, rows2, acc1_sh, acc2_sh, sem):
    c = lax.axis_index("c")
    s = lax.axis_index("s")
    zpt = SROWS // NS
    fpt = SEGB // NS

    def do_pass(p, carry):
        b = 2 * p + c
        pltpu.sync_copy(zeros_hbm, acc1_sh.at[pl.ds(s * zpt, zpt)])
        pltpu.sync_copy(zeros_hbm, acc2_sh.at[pl.ds(s * zpt, zpt)])
        plsc.subcore_barrier()
        for k in range(2):
            sw = 2 * s + k
            base = (b * NW + sw) * CAPW
            for q in range(CAPW // CH):
                pltpu.sync_copy(bsrc_hbm.at[pl.ds(base + q * CH, CH)],
                                sidx.at[q])
                pltpu.sync_copy(bldst_hbm.at[pl.ds(base + q * CH, CH)],
                                lidx.at[q])
            for q in range(CAPW // CH):
                pltpu.async_copy(x1_hbm.at[sidx.at[q]], rows1, sem).wait()
                pltpu.sync_copy(rows1, acc1_sh.at[lidx.at[q]], add=True)
                pltpu.async_copy(x2_hbm.at[sidx.at[q]], rows2, sem).wait()
                pltpu.sync_copy(rows2, acc2_sh.at[lidx.at[q]], add=True)
        plsc.subcore_barrier()
        pltpu.sync_copy(acc1_sh.at[pl.ds(s * fpt, fpt)],
                        out1_hbm.at[pl.ds(b * SEGB + s * fpt, fpt)])
        pltpu.sync_copy(acc2_sh.at[pl.ds(s * fpt, fpt)],
                        out2_hbm.at[pl.ds(b * SEGB + s * fpt, fpt)])
        plsc.subcore_barrier()
        return carry

    lax.fori_loop(0, NB // NC, do_pass, 0)


def _seg_big(x1, x2, bsrc, bldst, zeros):
    f = pl.kernel(
        _seg_big_body,
        out_type=(
            jax.ShapeDtypeStruct((E, D), jnp.float32),
            jax.ShapeDtypeStruct((E, D), jnp.float32),
        ),
        mesh=_make_mesh(),
        scratch_types=[
            pltpu.VMEM((CAPW // CH, CH), jnp.int32),
            pltpu.VMEM((CAPW // CH, CH), jnp.int32),
            pltpu.VMEM((CH, D), jnp.float32),
            pltpu.VMEM((CH, D), jnp.float32),
            pltpu.VMEM_SHARED((SROWS, D), jnp.float32),
            pltpu.VMEM_SHARED((SROWS, D), jnp.float32),
            pltpu.SemaphoreType.DMA,
        ],
    )
    return f(x1, x2, bsrc, bldst, zeros)


def _node_mlp_body(part_ref, x_ref, wn1_ref, bn1_ref, wn2_ref, bn2_ref,
                   h_ref):
    agg = part_ref[0] + part_ref[1]
    x = x_ref[...]
    t1 = _lrelu(_dot(agg, wn1_ref[...]) + bn1_ref[...]) + x
    t2 = _lrelu(_dot(agg, wn2_ref[...]) + bn2_ref[...]) + x
    h_ref[...] = jnp.concatenate([t1, t2], axis=1)


def _node_mlp(partial, x, wn1, bn1, wn2, bn2):
    blk = 1000
    grid = N_NODES // blk
    return pl.pallas_call(
        _node_mlp_body,
        grid=(grid,),
        in_specs=[
            pl.BlockSpec((NC, blk, D), lambda i: (0, i, 0)),
            pl.BlockSpec((blk, D), lambda i: (i, 0)),
            pl.BlockSpec((D, D), lambda i: (0, 0)),
            pl.BlockSpec((1, D), lambda i: (0, 0)),
            pl.BlockSpec((D, D), lambda i: (0, 0)),
            pl.BlockSpec((1, D), lambda i: (0, 0)),
        ],
        out_specs=pl.BlockSpec((blk, D2), lambda i: (i, 0)),
        out_shape=jax.ShapeDtypeStruct((N_NODES, D2), jnp.float32),
    )(partial, x, wn1, bn1, wn2, bn2)


def _edge_mlp_body(hs_ref, hd_ref, act_ref, ang_ref, gt_ref,
                   we1_ref, be1_ref, we2_ref, be2_ref,
                   e1_ref, e2_ref, sl_ref):
    i = pl.program_id(0)
    blk = hs_ref.shape[0]
    act = act_ref[...]
    ang = ang_ref[...]
    zpad = jnp.zeros((blk, 6), jnp.float32)
    hs = hs_ref[...]
    hd = hd_ref[...]
    e_in1 = jnp.concatenate([hs[:, :D], hd[:, :D], act, ang, zpad], axis=1)
    e1 = _lrelu(_dot(e_in1, we1_ref[...]) + be1_ref[...])
    e1_ref[...] = e1
    e_in2 = jnp.concatenate([hs[:, D:], hd[:, D:], act, ang, zpad], axis=1)
    e2 = _lrelu(_dot(e_in2, we2_ref[...]) + be2_ref[...])
    e2_ref[...] = e2
    rm1 = jnp.mean(e1, axis=1, keepdims=True)
    rm2 = jnp.mean(e2, axis=1, keepdims=True)
    s1 = jnp.sum((jnp.tanh(rm1) - gt_ref[...]) ** 2)
    s2 = jnp.sum((jnp.tanh(rm2) - gt_ref[...]) ** 2)
    lane = lax.broadcasted_iota(jnp.int32, (1, 128), 1)
    contrib = jnp.where(lane == 0, s1, 0.0) + jnp.where(lane == 1, s2, 0.0)

    @pl.when(i == 0)
    def _():
        sl_ref[...] = jnp.zeros_like(sl_ref)

    sl_ref[...] += contrib


def _edge_mlp(hs12, hd12, act, ang, gt, we1p, be1, we2p, be2):
    blk = 512
    grid = E // blk
    return pl.pallas_call(
        _edge_mlp_body,
        grid=(grid,),
        in_specs=[
            pl.BlockSpec((blk, D2), lambda i: (i, 0)),
            pl.BlockSpec((blk, D2), lambda i: (i, 0)),
            pl.BlockSpec((blk, 1), lambda i: (i, 0)),
            pl.BlockSpec((blk, 1), lambda i: (i, 0)),
            pl.BlockSpec((blk, 1), lambda i: (i, 0)),
            pl.BlockSpec((264, D), lambda i: (0, 0)),
            pl.BlockSpec((1, D), lambda i: (0, 0)),
            pl.BlockSpec((264, D), lambda i: (0, 0)),
            pl.BlockSpec((1, D), lambda i: (0, 0)),
        ],
        out_specs=[
            pl.BlockSpec((blk, D), lambda i: (i, 0)),
            pl.BlockSpec((blk, D), lambda i: (i, 0)),
            pl.BlockSpec((1, 128), lambda i: (0, 0)),
        ],
        out_shape=[
            jax.ShapeDtypeStruct((E, D), jnp.float32),
            jax.ShapeDtypeStruct((E, D), jnp.float32),
            jax.ShapeDtypeStruct((1, 128), jnp.float32),
        ],
    )(hs12, hd12, act, ang, gt, we1p, be1, we2p, be2)


def _gnn_step_body(x1_ref, x2_ref, a1_ref, a2_ref, w1_ref, b1_ref,
                   w2_ref, b2_ref, o1_ref, o2_ref):
    o1_ref[...] = _lrelu(_dot(x1_ref[...] + a1_ref[...], w1_ref[...])
                         + b1_ref[...])
    o2_ref[...] = _lrelu(_dot(x2_ref[...] + a2_ref[...], w2_ref[...])
                         + b2_ref[...])


def _gnn_step(x1, x2, a1, a2, w1, b1, w2, b2):
    blk = 512
    grid = E // blk
    bspec = pl.BlockSpec((blk, D), lambda i: (i, 0))
    wspec = pl.BlockSpec((D, D), lambda i: (0, 0))
    vspec = pl.BlockSpec((1, D), lambda i: (0, 0))
    return pl.pallas_call(
        _gnn_step_body,
        grid=(grid,),
        in_specs=[bspec, bspec, bspec, bspec, wspec, vspec, wspec, vspec],
        out_specs=[bspec, bspec],
        out_shape=[
            jax.ShapeDtypeStruct((E, D), jnp.float32),
            jax.ShapeDtypeStruct((E, D), jnp.float32),
        ],
    )(x1, x2, a1, a2, w1, b1, w2, b2)


def _pool_body(f1_ref, f2_ref, p1_ref, p2_ref, sq_ref):
    i = pl.program_id(0)
    blk = p1_ref.shape[0]
    f1 = f1_ref[...]
    f2 = f2_ref[...]
    p1_ref[...] = jnp.mean(f1.reshape(blk, SG_SIZE, D), axis=1)
    p2_ref[...] = jnp.mean(f2.reshape(blk, SG_SIZE, D), axis=1)
    s1 = jnp.sum(f1 * f1)
    s2 = jnp.sum(f2 * f2)
    lane = lax.broadcasted_iota(jnp.int32, (1, 128), 1)
    contrib = jnp.where(lane == 0, s1, 0.0) + jnp.where(lane == 1, s2, 0.0)

    @pl.when(i == 0)
    def _():
        sq_ref[...] = jnp.zeros_like(sq_ref)

    sq_ref[...] += contrib


def _pool(f1, f2):
    blk = 320
    grid = N_SUB // blk
    return pl.pallas_call(
        _pool_body,
        grid=(grid,),
        in_specs=[
            pl.BlockSpec((blk * SG_SIZE, D), lambda i: (i, 0)),
            pl.BlockSpec((blk * SG_SIZE, D), lambda i: (i, 0)),
        ],
        out_specs=[
            pl.BlockSpec((blk, D), lambda i: (i, 0)),
            pl.BlockSpec((blk, D), lambda i: (i, 0)),
            pl.BlockSpec((1, 128), lambda i: (0, 0)),
        ],
        out_shape=[
            jax.ShapeDtypeStruct((N_SUB, D), jnp.float32),
            jax.ShapeDtypeStruct((N_SUB, D), jnp.float32),
            jax.ShapeDtypeStruct((1, 128), jnp.float32),
        ],
    )(f1, f2)


def _mean_body(y_ref, st_ref):
    i = pl.program_id(0)
    s0 = jnp.sum(y_ref[...], axis=0, keepdims=True)
    contrib = jnp.concatenate(
        [s0, jnp.zeros((7, s0.shape[1]), jnp.float32)], axis=0)

    @pl.when(i == 0)
    def _():
        st_ref[...] = jnp.zeros_like(st_ref)

    st_ref[...] += contrib


def _var_body(y_ref, m_ref, st_ref):
    i = pl.program_id(0)
    m = m_ref[0:1, :] / jnp.float32(N_SUB)
    dv = y_ref[...] - m
    s0 = jnp.sum(dv * dv, axis=0, keepdims=True)
    contrib = jnp.concatenate(
        [s0, jnp.zeros((7, s0.shape[1]), jnp.float32)], axis=0)

    @pl.when(i == 0)
    def _():
        st_ref[...] = jnp.zeros_like(st_ref)

    st_ref[...] += contrib


def _mean(y):
    blk = 2000
    grid = N_SUB // blk
    wd = y.shape[1]
    return pl.pallas_call(
        _mean_body,
        grid=(grid,),
        in_specs=[pl.BlockSpec((blk, wd), lambda i: (i, 0))],
        out_specs=pl.BlockSpec((8, wd), lambda i: (0, 0)),
        out_shape=jax.ShapeDtypeStruct((8, wd), jnp.float32),
    )(y)


def _var(y, msum):
    blk = 2000
    grid = N_SUB // blk
    wd = y.shape[1]
    return pl.pallas_call(
        _var_body,
        grid=(grid,),
        in_specs=[
            pl.BlockSpec((blk, wd), lambda i: (i, 0)),
            pl.BlockSpec((8, wd), lambda i: (0, 0)),
        ],
        out_specs=pl.BlockSpec((8, wd), lambda i: (0, 0)),
        out_shape=jax.ShapeDtypeStruct((8, wd), jnp.float32),
    )(y, msum)


def _bn_mlp_body(y_ref, m_ref, v_ref, w_ref, g_ref, cb_ref, d_ref, o_ref):
    n = jnp.float32(N_SUB)
    m = m_ref[0:1, :] / n
    v = v_ref[0:1, :] / n
    z = (y_ref[...] - m) / jnp.sqrt(v + 1e-5) * g_ref[...] + cb_ref[...]
    z = _lrelu(z)
    o_ref[...] = _dot(z, w_ref[...]) + d_ref[...]


def _bn_mlp(y, msum, vsum, wd, gC, cC, dC):
    blk = 2000
    grid = N_SUB // blk
    win = y.shape[1]
    wout = wd.shape[1]
    return pl.pallas_call(
        _bn_mlp_body,
        grid=(grid,),
        in_specs=[
            pl.BlockSpec((blk, win), lambda i: (i, 0)),
            pl.BlockSpec((8, win), lambda i: (0, 0)),
            pl.BlockSpec((8, win), lambda i: (0, 0)),
            pl.BlockSpec((win, wout), lambda i: (0, 0)),
            pl.BlockSpec((1, win), lambda i: (0, 0)),
            pl.BlockSpec((1, win), lambda i: (0, 0)),
            pl.BlockSpec((1, wout), lambda i: (0, 0)),
        ],
        out_specs=pl.BlockSpec((blk, wout), lambda i: (i, 0)),
        out_shape=jax.ShapeDtypeStruct((N_SUB, wout), jnp.float32),
    )(y, msum, vsum, wd, gC, cC, dC)


def _value_head(pooled, p, pre):
    msum = _mean(pooled)
    vsum = _var(pooled, msum)
    y1 = _bn_mlp(pooled, msum, vsum, p[pre + '_W1'],
                 p[pre + '_g1'][None, :], p[pre + '_c1'][None, :],
                 p[pre + '_d1'][None, :])
    msum = _mean(y1)
    vsum = _var(y1, msum)
    y2 = _bn_mlp(y1, msum, vsum, p[pre + '_W2'],
                 p[pre + '_g2'][None, :], p[pre + '_c2'][None, :],
                 p[pre + '_d2'][None, :])
    msum = _mean(y2)
    vsum = _var(y2, msum)
    w3 = jnp.zeros((H, 128), jnp.float32).at[:, 0].set(p[pre + '_W3'][:, 0])
    d3 = jnp.zeros((1, 128), jnp.float32).at[0, 0].set(p[pre + '_d3'][0])
    qq = _bn_mlp(y2, msum, vsum, w3,
                 p[pre + '_g3'][None, :], p[pre + '_c3'][None, :], d3)
    return qq[:, 0]


def kernel(node_features, actions, edge_index, angles, sub_graphs,
           sep_subgraphs, gt_edges, post_input, params):
    p = params
    src3 = edge_index[0].reshape(NW, RPW, CH)
    dst3 = edge_index[1].reshape(NW, RPW, CH)
    sg3 = sub_graphs[0].reshape(NW, RPW, CH)
    sep = sep_subgraphs[0]
    gsrc = jnp.concatenate([sep[0], sep[1]])
    gdst = jnp.concatenate([sep[1], sep[0]])

    z_nodes = jnp.zeros((640, D), jnp.float32)
    z_big = jnp.zeros((SROWS // NS, D), jnp.float32)

    partial = _seg_nodes(node_features, src3, dst3, z_nodes)
    h12 = _node_mlp(partial, node_features, p['W1n'], p['b1n'][None, :],
                    p['W2n'], p['b2n'][None, :])

    hs12 = _gather_rows(h12, src3)
    hd12 = _gather_rows(h12, dst3)
    we1p = jnp.concatenate([p['W1e'], jnp.zeros((6, D), jnp.float32)], axis=0)
    we2p = jnp.concatenate([p['W2e'], jnp.zeros((6, D), jnp.float32)], axis=0)
    e1, e2, sl_edge = _edge_mlp(
        hs12, hd12, actions[:, None], angles[:, None], gt_edges[:, None],
        we1p, p['b1e'][None, :], we2p, p['b2e'][None, :])
    sl1 = sl_edge[0, 0] / E
    sl2 = sl_edge[0, 1] / E

    bid = gdst // SEGB
    wid = jnp.arange(E, dtype=jnp.int32) // (E // NW)
    slot = bid * NW + wid
    order = jnp.argsort(slot, stable=True)
    sslot = slot[order]
    start = jnp.searchsorted(sslot, jnp.arange(NB * NW, dtype=jnp.int32))
    pos = jnp.arange(E, dtype=jnp.int32) - start[sslot]
    tgt = jnp.where(pos < CAPW, sslot * CAPW + pos, NB * NW * CAPW)
    bsrc = jnp.zeros((NB * NW * CAPW + CAPW,), jnp.int32).at[tgt].set(
        gsrc[order], mode='drop')
    bldst = jnp.full((NB * NW * CAPW + CAPW,), TRASH, jnp.int32).at[tgt].set(
        (gdst - bid * SEGB)[order], mode='drop')

    f1 = _gather_rows(e1, sg3)
    f2 = _gather_rows(e2, sg3)
    for _ in range(SG_SIZE // 2):
        a1, a2 = _seg_big(f1, f2, bsrc, bldst, z_big)
        f1, f2 = _gnn_step(f1, f2, a1, a2, p['Wg1'], p['bg1'][None, :],
                           p['Wg2'], p['bg2'][None, :])

    p1, p2, sq = _pool(f1, f2)
    sl3 = sq[0, 0] / (E * D)
    sl4 = sq[0, 1] / (E * D)

    q1 = _value_head(p1, p, 'v1')
    q2 = _value_head(p2, p, 'v2')
    side_loss = (sl1 + sl2 + sl3 + sl4) / 4.0
    return q1, q2, side_loss
